# Initial kernel scaffold; baseline (speedup 1.0000x reference)
import jax
import jax.numpy as jnp
from jax.experimental import pallas as pl


def kernel(x, edge_index, edge_attr, W_lin, b_lin, W_e0, b_e0, ln_gamma, ln_beta, W_e3, b_e3, W_proj, b_proj, bias_p):
    raise NotImplementedError("write your pallas kernel here")


# speedup vs baseline: 2.1541x; 2.1541x over previous
import functools

import jax
import jax.numpy as jnp
from jax import lax
from jax.experimental import pallas as pl
from jax.experimental.pallas import tpu as pltpu, tpu_sc as plsc

N = 10000
E = 320000
D = 128
DE = 16

NC = 2
NS = 16
NW = NC * NS
EW = E // NW
CE = 80
NCHUNK = EW // CE
N_PAD = 10240
ROWS_PER_TILE = N_PAD // NS

_DOT_T = (((1,), (1,)), ((), ()))


def _node_body(x_ref, wlin_ref, blin_ref, wp0_ref, wp1_ref, btot_ref,
               wp2_ref, we3_ref, be3_ref, y_ref, baseh_ref, wc_ref, c2_ref):
    x2 = lax.dot_general(x_ref[...], wlin_ref[...], _DOT_T,
                         preferred_element_type=jnp.float32,
                         precision=lax.Precision.HIGHEST) + blin_ref[...]
    y_ref[...] = lax.dot_general(x2, wp1_ref[...], _DOT_T,
                                 preferred_element_type=jnp.float32,
                                 precision=lax.Precision.HIGHEST)
    baseh_ref[...] = 0.5 * (
        lax.dot_general(x2, wp0_ref[...], _DOT_T,
                        preferred_element_type=jnp.float32,
                        precision=lax.Precision.HIGHEST) + btot_ref[...])
    wc_ref[...] = lax.dot_general(wp2_ref[...], we3_ref[...],
                                  (((1,), (0,)), ((), ())),
                                  preferred_element_type=jnp.float32,
                                  precision=lax.Precision.HIGHEST)
    c2_ref[...] = lax.dot_general(be3_ref[...], wp2_ref[...], _DOT_T,
                                  preferred_element_type=jnp.float32,
                                  precision=lax.Precision.HIGHEST)


def _edge_body(ea_ref, we0_ref, be0_ref, gamma_ref, beta_ref, wc_ref, c2_ref,
               z_ref):
    a = lax.dot_general(ea_ref[...], we0_ref[...], _DOT_T,
                        preferred_element_type=jnp.float32,
                        precision=lax.Precision.HIGHEST) + be0_ref[...]
    e0 = jnp.maximum(a, 0.0)
    mu = jnp.mean(e0, axis=-1, keepdims=True)
    d0 = e0 - mu
    var = jnp.mean(d0 * d0, axis=-1, keepdims=True)
    ln = d0 * lax.rsqrt(var + 1e-5) * gamma_ref[...] + beta_ref[...]
    z_ref[...] = lax.dot_general(ln, wc_ref[...], _DOT_T,
                                 preferred_element_type=jnp.float32,
                                 precision=lax.Precision.HIGHEST) + c2_ref[...]


def _sc_body(y_hbm, z_hbm, src_hbm, dst_hbm, baseh_hbm, out_hbm,
             src_v, dst_v, y_v, z_v, acc, sem):
    cid = lax.axis_index("c")
    sid = lax.axis_index("s")
    row0 = sid * ROWS_PER_TILE
    pltpu.sync_copy(baseh_hbm.at[pl.ds(row0, ROWS_PER_TILE)],
                    acc.at[pl.ds(row0, ROWS_PER_TILE)])
    plsc.subcore_barrier()

    edge0 = (cid * NS + sid) * EW

    def step(k, carry):
        b = edge0 + k * CE
        pltpu.sync_copy(src_hbm.at[pl.ds(b, CE)], src_v)
        pltpu.sync_copy(dst_hbm.at[pl.ds(b, CE)], dst_v)
        pltpu.async_copy(y_hbm.at[src_v], y_v, sem).wait()
        pltpu.sync_copy(z_hbm.at[pl.ds(b, CE)], z_v)
        pltpu.sync_copy(y_v, acc.at[dst_v], add=True)
        pltpu.sync_copy(z_v, acc.at[dst_v], add=True)
        return carry

    lax.fori_loop(0, NCHUNK, step, 0)
    plsc.subcore_barrier()
    pltpu.sync_copy(acc.at[pl.ds(row0, ROWS_PER_TILE)],
                    out_hbm.at[cid, pl.ds(row0, ROWS_PER_TILE)])


def _combine_body(a_ref, b_ref, o_ref):
    o_ref[...] = a_ref[...] + b_ref[...]


def kernel(x, edge_index, edge_attr, W_lin, b_lin, W_e0, b_e0, ln_gamma,
           ln_beta, W_e3, b_e3, W_proj, b_proj, bias_p):
    src = edge_index[0].astype(jnp.int32)
    dst = edge_index[1].astype(jnp.int32)
    Wp0 = W_proj[:, 0:D]
    Wp1 = W_proj[:, D:2 * D]
    Wp2 = W_proj[:, 2 * D:3 * D]
    b_tot = (b_proj + bias_p).reshape(1, D)

    y, base_half, Wc, c2 = pl.pallas_call(
        _node_body,
        out_shape=(
            jax.ShapeDtypeStruct((N, D), jnp.float32),
            jax.ShapeDtypeStruct((N, D), jnp.float32),
            jax.ShapeDtypeStruct((D, D), jnp.float32),
            jax.ShapeDtypeStruct((1, D), jnp.float32),
        ),
    )(x, W_lin, b_lin.reshape(1, D), Wp0, Wp1, b_tot, Wp2, W_e3,
      b_e3.reshape(1, D))

    EB = 2000
    z = pl.pallas_call(
        _edge_body,
        grid=(E // EB,),
        in_specs=[
            pl.BlockSpec((EB, DE), lambda i: (i, 0)),
            pl.BlockSpec((D, DE), lambda i: (0, 0)),
            pl.BlockSpec((1, D), lambda i: (0, 0)),
            pl.BlockSpec((1, D), lambda i: (0, 0)),
            pl.BlockSpec((1, D), lambda i: (0, 0)),
            pl.BlockSpec((D, D), lambda i: (0, 0)),
            pl.BlockSpec((1, D), lambda i: (0, 0)),
        ],
        out_specs=pl.BlockSpec((EB, D), lambda i: (i, 0)),
        out_shape=jax.ShapeDtypeStruct((E, D), jnp.float32),
    )(edge_attr, W_e0, b_e0.reshape(1, D), ln_gamma.reshape(1, D),
      ln_beta.reshape(1, D), Wc, c2)

    base_half_pad = jnp.pad(base_half, ((0, N_PAD - N), (0, 0)))
    mesh = plsc.VectorSubcoreMesh(core_axis_name="c", subcore_axis_name="s",
                                  num_cores=NC, num_subcores=NS)
    partials = pl.kernel(
        _sc_body,
        out_type=jax.ShapeDtypeStruct((NC, N_PAD, D), jnp.float32),
        mesh=mesh,
        scratch_types=[
            pltpu.VMEM((CE,), jnp.int32),
            pltpu.VMEM((CE,), jnp.int32),
            pltpu.VMEM((CE, D), jnp.float32),
            pltpu.VMEM((CE, D), jnp.float32),
            pltpu.VMEM_SHARED((N_PAD, D), jnp.float32),
            pltpu.SemaphoreType.DMA,
        ],
    )(y, z, src, dst, base_half_pad)

    NB = 1000
    h = pl.pallas_call(
        _combine_body,
        grid=(N // NB,),
        in_specs=[
            pl.BlockSpec((NB, D), lambda i: (i, 0)),
            pl.BlockSpec((NB, D), lambda i: (i, 0)),
        ],
        out_specs=pl.BlockSpec((NB, D), lambda i: (i, 0)),
        out_shape=jax.ShapeDtypeStruct((N, D), jnp.float32),
    )(partials[0], partials[1])
    return h


# speedup vs baseline: 3.1506x; 1.4626x over previous
import functools

import jax
import jax.numpy as jnp
from jax import lax
from jax.experimental import pallas as pl
from jax.experimental.pallas import tpu as pltpu, tpu_sc as plsc

N = 10000
E = 320000
D = 128
DE = 16

NC = 2
NS = 16
NW = NC * NS
EW = E // NW
CE = 80
NCHUNK = EW // CE
N_PAD = 10240
ROWS_PER_TILE = N_PAD // NS

_DOT_T = (((1,), (1,)), ((), ()))


def _node_body(x_ref, wlin_ref, blin_ref, wp0_ref, wp1_ref, btot_ref,
               wp2_ref, we3_ref, be3_ref, y_ref, baseh_ref, wc_ref, c2_ref):
    x2 = lax.dot_general(x_ref[...], wlin_ref[...], _DOT_T,
                         preferred_element_type=jnp.float32) + blin_ref[...]
    y_ref[...] = lax.dot_general(x2, wp1_ref[...], _DOT_T,
                                 preferred_element_type=jnp.float32)
    baseh_ref[...] = 0.5 * (
        lax.dot_general(x2, wp0_ref[...], _DOT_T,
                        preferred_element_type=jnp.float32) + btot_ref[...])
    wc_ref[...] = lax.dot_general(wp2_ref[...], we3_ref[...],
                                  (((1,), (0,)), ((), ())),
                                  preferred_element_type=jnp.float32)
    c2_ref[...] = lax.dot_general(be3_ref[...], wp2_ref[...], _DOT_T,
                                  preferred_element_type=jnp.float32)


def _edge_body(ea_ref, we0_ref, be0_ref, gamma_ref, beta_ref, wc_ref, c2_ref,
               z_ref):
    a = lax.dot_general(ea_ref[...], we0_ref[...], _DOT_T,
                        preferred_element_type=jnp.float32) + be0_ref[...]
    e0 = jnp.maximum(a, 0.0)
    mu = jnp.mean(e0, axis=-1, keepdims=True)
    d0 = e0 - mu
    var = jnp.mean(d0 * d0, axis=-1, keepdims=True)
    ln = d0 * lax.rsqrt(var + 1e-5) * gamma_ref[...] + beta_ref[...]
    z_ref[...] = lax.dot_general(ln, wc_ref[...], _DOT_T,
                                 preferred_element_type=jnp.float32) + c2_ref[...]


def _sc_body(y_hbm, z_hbm, ei_hbm, baseh_hbm, out_hbm,
             src_v, dst_v, y_v, z_v, acc, sem):
    cid = lax.axis_index("c")
    sid = lax.axis_index("s")
    row0 = sid * ROWS_PER_TILE
    pltpu.sync_copy(baseh_hbm.at[pl.ds(row0, ROWS_PER_TILE)],
                    acc.at[pl.ds(row0, ROWS_PER_TILE)])
    plsc.subcore_barrier()

    edge0 = (cid * NS + sid) * EW

    def step(k, carry):
        b = edge0 + k * CE
        pltpu.sync_copy(ei_hbm.at[pl.ds(b, CE)], src_v)
        pltpu.sync_copy(ei_hbm.at[pl.ds(E + b, CE)], dst_v)
        pltpu.async_copy(y_hbm.at[src_v], y_v, sem).wait()
        pltpu.sync_copy(z_hbm.at[pl.ds(b, CE)], z_v)
        pltpu.sync_copy(y_v, acc.at[dst_v], add=True)
        pltpu.sync_copy(z_v, acc.at[dst_v], add=True)
        return carry

    lax.fori_loop(0, NCHUNK, step, 0)
    plsc.subcore_barrier()
    pltpu.sync_copy(acc.at[pl.ds(row0, ROWS_PER_TILE)],
                    out_hbm.at[cid, pl.ds(row0, ROWS_PER_TILE)])


def _combine_body(a_ref, b_ref, o_ref):
    o_ref[...] = a_ref[...] + b_ref[...]


def kernel(x, edge_index, edge_attr, W_lin, b_lin, W_e0, b_e0, ln_gamma,
           ln_beta, W_e3, b_e3, W_proj, b_proj, bias_p):
    ei = edge_index.astype(jnp.int32).reshape(2 * E)
    Wp0 = W_proj[:, 0:D]
    Wp1 = W_proj[:, D:2 * D]
    Wp2 = W_proj[:, 2 * D:3 * D]
    b_tot = (b_proj + bias_p).reshape(1, D)

    y, base_half, Wc, c2 = pl.pallas_call(
        _node_body,
        out_shape=(
            jax.ShapeDtypeStruct((N, D), jnp.float32),
            jax.ShapeDtypeStruct((N, D), jnp.float32),
            jax.ShapeDtypeStruct((D, D), jnp.float32),
            jax.ShapeDtypeStruct((1, D), jnp.float32),
        ),
    )(x, W_lin, b_lin.reshape(1, D), Wp0, Wp1, b_tot, Wp2, W_e3,
      b_e3.reshape(1, D))

    EB = 2000
    z = pl.pallas_call(
        _edge_body,
        grid=(E // EB,),
        in_specs=[
            pl.BlockSpec((EB, DE), lambda i: (i, 0)),
            pl.BlockSpec((D, DE), lambda i: (0, 0)),
            pl.BlockSpec((1, D), lambda i: (0, 0)),
            pl.BlockSpec((1, D), lambda i: (0, 0)),
            pl.BlockSpec((1, D), lambda i: (0, 0)),
            pl.BlockSpec((D, D), lambda i: (0, 0)),
            pl.BlockSpec((1, D), lambda i: (0, 0)),
        ],
        out_specs=pl.BlockSpec((EB, D), lambda i: (i, 0)),
        out_shape=jax.ShapeDtypeStruct((E, D), jnp.float32),
    )(edge_attr, W_e0, b_e0.reshape(1, D), ln_gamma.reshape(1, D),
      ln_beta.reshape(1, D), Wc, c2)

    base_half_pad = jnp.pad(base_half, ((0, N_PAD - N), (0, 0)))
    mesh = plsc.VectorSubcoreMesh(core_axis_name="c", subcore_axis_name="s",
                                  num_cores=NC, num_subcores=NS)
    partials = pl.kernel(
        _sc_body,
        out_type=jax.ShapeDtypeStruct((NC, N_PAD, D), jnp.float32),
        mesh=mesh,
        scratch_types=[
            pltpu.VMEM((CE,), jnp.int32),
            pltpu.VMEM((CE,), jnp.int32),
            pltpu.VMEM((CE, D), jnp.float32),
            pltpu.VMEM((CE, D), jnp.float32),
            pltpu.VMEM_SHARED((N_PAD, D), jnp.float32),
            pltpu.SemaphoreType.DMA,
        ],
    )(y, z, ei, base_half_pad)

    NB = 1000
    h = pl.pallas_call(
        _combine_body,
        grid=(N // NB,),
        in_specs=[
            pl.BlockSpec((NB, D), lambda i: (i, 0)),
            pl.BlockSpec((NB, D), lambda i: (i, 0)),
        ],
        out_specs=pl.BlockSpec((NB, D), lambda i: (i, 0)),
        out_shape=jax.ShapeDtypeStruct((N, D), jnp.float32),
    )(partials[0], partials[1])
    return h


# speedup vs baseline: 5.2505x; 1.6665x over previous
import functools

import jax
import jax.numpy as jnp
from jax import lax
from jax.experimental import pallas as pl
from jax.experimental.pallas import tpu as pltpu, tpu_sc as plsc

N = 10000
E = 320000
D = 128
DE = 16

NC = 2
NS = 16
NW = NC * NS
EW = E // NW
CE = 80
NCHUNK = EW // CE
NPAIR = (NCHUNK + 1) // 2
N_PAD = 10240
ROWS_PER_TILE = N_PAD // NS

_DOT_T = (((1,), (1,)), ((), ()))


def _node_body(x_ref, wlin_ref, blin_ref, wp0_ref, wp1_ref, btot_ref,
               wp2_ref, we3_ref, be3_ref, y_ref, baseh_ref, wc_ref, c2_ref):
    x2 = lax.dot_general(x_ref[...], wlin_ref[...], _DOT_T,
                         preferred_element_type=jnp.float32) + blin_ref[...]
    y_ref[...] = lax.dot_general(x2, wp1_ref[...], _DOT_T,
                                 preferred_element_type=jnp.float32)
    baseh_ref[...] = 0.5 * (
        lax.dot_general(x2, wp0_ref[...], _DOT_T,
                        preferred_element_type=jnp.float32) + btot_ref[...])
    wc_ref[...] = lax.dot_general(wp2_ref[...], we3_ref[...],
                                  (((1,), (0,)), ((), ())),
                                  preferred_element_type=jnp.float32)
    c2_ref[...] = lax.dot_general(be3_ref[...], wp2_ref[...], _DOT_T,
                                  preferred_element_type=jnp.float32)


def _edge_body(ea_ref, we0_ref, be0_ref, gamma_ref, beta_ref, wc_ref, c2_ref,
               z_ref):
    a = lax.dot_general(ea_ref[...], we0_ref[...], _DOT_T,
                        preferred_element_type=jnp.float32) + be0_ref[...]
    e0 = jnp.maximum(a, 0.0)
    mu = jnp.mean(e0, axis=-1, keepdims=True)
    d0 = e0 - mu
    var = jnp.mean(d0 * d0, axis=-1, keepdims=True)
    ln = d0 * lax.rsqrt(var + 1e-5) * gamma_ref[...] + beta_ref[...]
    z_ref[...] = lax.dot_general(ln, wc_ref[...], _DOT_T,
                                 preferred_element_type=jnp.float32) + c2_ref[...]


def _sc_gather_body(y_hbm, ei_hbm, baseh_hbm, out_hbm,
                    src_v, dst_v, row_v, acc, sems):
    cid = lax.axis_index("c")
    sid = lax.axis_index("s")
    wid = cid * NS + sid
    row0 = sid * ROWS_PER_TILE
    pltpu.sync_copy(baseh_hbm.at[pl.ds(row0, ROWS_PER_TILE)],
                    acc.at[pl.ds(row0, ROWS_PER_TILE)])
    plsc.subcore_barrier()

    edge0 = wid * EW
    pltpu.sync_copy(ei_hbm.at[pl.ds(edge0, CE)], src_v.at[0])
    pltpu.sync_copy(ei_hbm.at[pl.ds(E + edge0, CE)], dst_v.at[0])
    pltpu.async_copy(y_hbm.at[src_v.at[0]], row_v.at[0], sems[0])

    def pair(p, carry):
        for b in range(2):
            k = 2 * p + b
            nb = 1 - b

            @pl.when(k < NCHUNK)
            def _():
                @pl.when(k + 1 < NCHUNK)
                def _():
                    nb_off = edge0 + (k + 1) * CE
                    pltpu.sync_copy(ei_hbm.at[pl.ds(nb_off, CE)],
                                    src_v.at[nb])
                    pltpu.sync_copy(ei_hbm.at[pl.ds(E + nb_off, CE)],
                                    dst_v.at[nb])
                    pltpu.async_copy(y_hbm.at[src_v.at[nb]], row_v.at[nb],
                                     sems[nb])
                pltpu.make_async_copy(y_hbm.at[src_v.at[b]], row_v.at[b],
                                      sems[b]).wait()
                pltpu.sync_copy(row_v.at[b], acc.at[dst_v.at[b]], add=True)
        return carry

    lax.fori_loop(0, NPAIR, pair, 0)
    plsc.subcore_barrier()
    pltpu.sync_copy(acc.at[pl.ds(row0, ROWS_PER_TILE)],
                    out_hbm.at[cid, pl.ds(row0, ROWS_PER_TILE)])


def _sc_scatter_body(z_hbm, ei_hbm, part_hbm, out_hbm,
                     dst_v, row_v, acc, sems):
    cid = lax.axis_index("c")
    sid = lax.axis_index("s")
    wid = cid * NS + sid
    row0 = sid * ROWS_PER_TILE
    pltpu.sync_copy(part_hbm.at[cid, pl.ds(row0, ROWS_PER_TILE)],
                    acc.at[pl.ds(row0, ROWS_PER_TILE)])
    plsc.subcore_barrier()

    edge0 = wid * EW
    pltpu.sync_copy(ei_hbm.at[pl.ds(E + edge0, CE)], dst_v.at[0])
    pltpu.async_copy(z_hbm.at[pl.ds(edge0, CE)], row_v.at[0], sems[0])

    def pair(p, carry):
        for b in range(2):
            k = 2 * p + b
            nb = 1 - b

            @pl.when(k < NCHUNK)
            def _():
                @pl.when(k + 1 < NCHUNK)
                def _():
                    pltpu.sync_copy(
                        ei_hbm.at[pl.ds(E + edge0 + (k + 1) * CE, CE)],
                        dst_v.at[nb])
                    pltpu.async_copy(
                        z_hbm.at[pl.ds(edge0 + (k + 1) * CE, CE)],
                        row_v.at[nb], sems[nb])
                pltpu.make_async_copy(z_hbm.at[pl.ds(edge0 + k * CE, CE)],
                                      row_v.at[b], sems[b]).wait()
                pltpu.sync_copy(row_v.at[b], acc.at[dst_v.at[b]], add=True)
        return carry

    lax.fori_loop(0, NPAIR, pair, 0)
    plsc.subcore_barrier()
    pltpu.sync_copy(acc.at[pl.ds(row0, ROWS_PER_TILE)],
                    out_hbm.at[cid, pl.ds(row0, ROWS_PER_TILE)])


def _combine_body(a_ref, b_ref, o_ref):
    o_ref[...] = a_ref[...] + b_ref[...]


def kernel(x, edge_index, edge_attr, W_lin, b_lin, W_e0, b_e0, ln_gamma,
           ln_beta, W_e3, b_e3, W_proj, b_proj, bias_p):
    ei = edge_index.astype(jnp.int32).reshape(2 * E)
    Wp0 = W_proj[:, 0:D]
    Wp1 = W_proj[:, D:2 * D]
    Wp2 = W_proj[:, 2 * D:3 * D]
    b_tot = (b_proj + bias_p).reshape(1, D)

    y, base_half, Wc, c2 = pl.pallas_call(
        _node_body,
        out_shape=(
            jax.ShapeDtypeStruct((N, D), jnp.float32),
            jax.ShapeDtypeStruct((N, D), jnp.float32),
            jax.ShapeDtypeStruct((D, D), jnp.float32),
            jax.ShapeDtypeStruct((1, D), jnp.float32),
        ),
    )(x, W_lin, b_lin.reshape(1, D), Wp0, Wp1, b_tot, Wp2, W_e3,
      b_e3.reshape(1, D))

    EB = 2000
    z = pl.pallas_call(
        _edge_body,
        grid=(E // EB,),
        in_specs=[
            pl.BlockSpec((EB, DE), lambda i: (i, 0)),
            pl.BlockSpec((D, DE), lambda i: (0, 0)),
            pl.BlockSpec((1, D), lambda i: (0, 0)),
            pl.BlockSpec((1, D), lambda i: (0, 0)),
            pl.BlockSpec((1, D), lambda i: (0, 0)),
            pl.BlockSpec((D, D), lambda i: (0, 0)),
            pl.BlockSpec((1, D), lambda i: (0, 0)),
        ],
        out_specs=pl.BlockSpec((EB, D), lambda i: (i, 0)),
        out_shape=jax.ShapeDtypeStruct((E, D), jnp.float32),
    )(edge_attr, W_e0, b_e0.reshape(1, D), ln_gamma.reshape(1, D),
      ln_beta.reshape(1, D), Wc, c2)

    base_half_pad = jnp.pad(base_half, ((0, N_PAD - N), (0, 0)))
    mesh = plsc.VectorSubcoreMesh(core_axis_name="c", subcore_axis_name="s",
                                  num_cores=NC, num_subcores=NS)
    part1 = pl.kernel(
        _sc_gather_body,
        out_type=jax.ShapeDtypeStruct((NC, N_PAD, D), jnp.float32),
        mesh=mesh,
        scratch_types=[
            pltpu.VMEM((2, CE), jnp.int32),
            pltpu.VMEM((2, CE), jnp.int32),
            pltpu.VMEM((2, CE, D), jnp.float32),
            pltpu.VMEM_SHARED((N_PAD, D), jnp.float32),
            (pltpu.SemaphoreType.DMA, pltpu.SemaphoreType.DMA),
        ],
    )(y, ei, base_half_pad)

    partials = pl.kernel(
        _sc_scatter_body,
        out_type=jax.ShapeDtypeStruct((NC, N_PAD, D), jnp.float32),
        mesh=mesh,
        scratch_types=[
            pltpu.VMEM((2, CE), jnp.int32),
            pltpu.VMEM((2, CE, D), jnp.float32),
            pltpu.VMEM_SHARED((N_PAD, D), jnp.float32),
            (pltpu.SemaphoreType.DMA, pltpu.SemaphoreType.DMA),
        ],
    )(z, ei, part1)

    NB = 1000
    h = pl.pallas_call(
        _combine_body,
        grid=(N // NB,),
        in_specs=[
            pl.BlockSpec((NB, D), lambda i: (i, 0)),
            pl.BlockSpec((NB, D), lambda i: (i, 0)),
        ],
        out_specs=pl.BlockSpec((NB, D), lambda i: (i, 0)),
        out_shape=jax.ShapeDtypeStruct((N, D), jnp.float32),
    )(partials[0], partials[1])
    return h
